# baseline (speedup 1.0000x reference)
import functools

import jax
import jax.numpy as jnp
from jax.experimental import pallas as pl


def _body(x_ref, m1_ref, m2_ref, aux_ref, xh_ref, idx_ref, n_ref, *, nlev):
    xb = x_ref[...]
    off = aux_ref[0:1, :]
    step = aux_ref[1:2, :]
    cmin = aux_ref[2:3, :]

    dot = functools.partial(
        jax.lax.dot_general,
        dimension_numbers=(((1,), (0,)), ((), ())),
        preferred_element_type=jnp.float32,
        precision=jax.lax.Precision.HIGHEST,
    )

    sq = xb * xb
    s2 = dot(sq, aux_ref[3:4, :].T)
    nrm = jnp.maximum(jnp.sqrt(s2), 1e-8)
    rec = 1.0 / nrm

    y = dot(xb, m1_ref[...])
    t = y * rec + off
    r = jnp.clip(jnp.round(t), 0.0, float(nlev - 1))
    idx_ref[...] = r.astype(jnp.int32)

    q = r * step + cmin
    xh_ref[...] = dot(q, m2_ref[...]) * nrm
    n_ref[...] = nrm


def kernel(x, centroids, rot2):
    d = x.shape[-1]
    n_groups = rot2.shape[0]
    assert n_groups * 2 == d, "kernel assumes no padding (d even)"
    nlev = centroids.shape[0]

    batch_shape = x.shape[:-1]
    rows = 1
    for dim in batch_shape:
        rows *= dim
    xf = x.reshape(rows, d)

    c = rot2[:, 0]
    s = rot2[:, 1]
    cmin = centroids[0]
    cmax = centroids[-1]
    sc = (nlev - 1) / (cmax - cmin)
    step = (cmax - cmin) / (nlev - 1)
    off = -cmin * sc

    e = jnp.arange(n_groups) * 2
    m1 = jnp.zeros((d, d), jnp.float32)
    m1 = m1.at[(e, e)].set(sc * c).at[(e + 1, e + 1)].set(sc * c)
    m1 = m1.at[(e + 1, e)].set(-sc * s).at[(e, e + 1)].set(sc * s)
    m2 = jnp.zeros((d, d), jnp.float32)
    m2 = m2.at[(e, e)].set(c).at[(e + 1, e + 1)].set(c)
    m2 = m2.at[(e + 1, e)].set(s).at[(e, e + 1)].set(-s)

    fill = lambda v: jnp.full((d,), v, dtype=jnp.float32)
    aux = jnp.stack(
        [fill(off), fill(step), fill(cmin), fill(1.0)]
        + [jnp.zeros((d,), jnp.float32)] * 4,
        axis=0,
    )

    BR = 4096
    assert rows % BR == 0
    grid = (rows // BR,)

    xh, idx, nrm = pl.pallas_call(
        functools.partial(_body, nlev=nlev),
        grid=grid,
        in_specs=[
            pl.BlockSpec((BR, d), lambda i: (i, 0)),
            pl.BlockSpec((d, d), lambda i: (0, 0)),
            pl.BlockSpec((d, d), lambda i: (0, 0)),
            pl.BlockSpec((8, d), lambda i: (0, 0)),
        ],
        out_specs=[
            pl.BlockSpec((BR, d), lambda i: (i, 0)),
            pl.BlockSpec((BR, d), lambda i: (i, 0)),
            pl.BlockSpec((BR, 1), lambda i: (i, 0)),
        ],
        out_shape=[
            jax.ShapeDtypeStruct((rows, d), jnp.float32),
            jax.ShapeDtypeStruct((rows, d), jnp.int32),
            jax.ShapeDtypeStruct((rows, 1), jnp.float32),
        ],
    )(xf, m1, m2, aux)

    return (xh.reshape(x.shape), idx.reshape(x.shape),
            nrm.reshape(batch_shape))


# speedup vs baseline: 4.5318x; 4.5318x over previous
import functools

import jax
import jax.numpy as jnp
from jax.experimental import pallas as pl


def _body(x_ref, coef_ref, xh_ref, idx_ref, n_ref, *, nlev):
    xb = x_ref[...]
    cf1 = coef_ref[:, 0:1]
    a1 = coef_ref[:, 1:2]
    b1 = coef_ref[:, 2:3]
    cf2 = coef_ref[:, 3:4]
    a2 = coef_ref[:, 4:5]
    b2 = coef_ref[:, 5:6]
    off = coef_ref[0:1, 6:7]
    step = coef_ref[0:1, 7:8]
    cmin = coef_ref[0:1, 8:9]

    s2 = jnp.sum(xb * xb, axis=0, keepdims=True)
    nrm = jnp.maximum(jnp.sqrt(s2), 1e-8)
    rec = 1.0 / nrm

    xl = jnp.roll(xb, -1, axis=0)
    xr = jnp.roll(xb, 1, axis=0)
    vr = cf1 * xb + a1 * xl + b1 * xr
    t = vr * rec + off
    r = jnp.clip(jnp.round(t), 0.0, float(nlev - 1))
    idx_ref[...] = r.astype(jnp.int32)

    q = r * step + cmin
    ql = jnp.roll(q, -1, axis=0)
    qr = jnp.roll(q, 1, axis=0)
    xh_ref[...] = (cf2 * q + a2 * ql + b2 * qr) * nrm
    n_ref[...] = nrm.reshape(n_ref.shape)


def kernel(x, centroids, rot2):
    d = x.shape[-1]
    n_groups = rot2.shape[0]
    assert n_groups * 2 == d, "kernel assumes no padding (d even)"
    nlev = centroids.shape[0]
    b, t = x.shape[0], x.shape[1]

    xt = jnp.transpose(x, (0, 2, 1)).reshape(b * d, t)

    c = rot2[:, 0]
    s = rot2[:, 1]
    z = jnp.zeros_like(s)
    cfull = jnp.stack([c, c], axis=-1).reshape(-1)
    a1 = jnp.stack([-s, z], axis=-1).reshape(-1)
    b1 = jnp.stack([z, s], axis=-1).reshape(-1)
    a2 = jnp.stack([s, z], axis=-1).reshape(-1)
    b2 = jnp.stack([z, -s], axis=-1).reshape(-1)

    cmin = centroids[0]
    cmax = centroids[-1]
    sc = (nlev - 1) / (cmax - cmin)
    step = (cmax - cmin) / (nlev - 1)
    off = -cmin * sc

    fill = lambda v: jnp.full((d,), v, dtype=jnp.float32)
    cols = [cfull * sc, a1 * sc, b1 * sc, cfull, a2, b2,
            fill(off), fill(step), fill(cmin)]
    while len(cols) < 16:
        cols.append(jnp.zeros((d,), jnp.float32))
    coef = jnp.stack(cols, axis=1)

    BT = 8192
    assert t % BT == 0
    grid = (b, t // BT)

    xh_t, idx_t, nrm = pl.pallas_call(
        functools.partial(_body, nlev=nlev),
        grid=grid,
        in_specs=[
            pl.BlockSpec((d, BT), lambda i, j: (i, j)),
            pl.BlockSpec((d, 16), lambda i, j: (0, 0)),
        ],
        out_specs=[
            pl.BlockSpec((d, BT), lambda i, j: (i, j)),
            pl.BlockSpec((d, BT), lambda i, j: (i, j)),
            pl.BlockSpec((1, 1, BT), lambda i, j: (i, 0, j)),
        ],
        out_shape=[
            jax.ShapeDtypeStruct((b * d, t), jnp.float32),
            jax.ShapeDtypeStruct((b * d, t), jnp.int32),
            jax.ShapeDtypeStruct((b, 1, t), jnp.float32),
        ],
    )(xt, coef)

    xh = jnp.transpose(xh_t.reshape(b, d, t), (0, 2, 1))
    idx = jnp.transpose(idx_t.reshape(b, d, t), (0, 2, 1))
    return xh, idx, nrm.reshape(b, t)
